# Initial kernel scaffold; baseline (speedup 1.0000x reference)
import jax
import jax.numpy as jnp
from jax.experimental import pallas as pl


def kernel(x, edge_index, W1, b1, W2, b2, W3, b3, W4, b4, W5, b5, W6, b6):
    raise NotImplementedError("write your pallas kernel here")


# speedup vs baseline: 15.4077x; 15.4077x over previous
import functools

import jax
import jax.numpy as jnp
from jax import lax
from jax.experimental import pallas as pl
from jax.experimental.pallas import tpu as pltpu
from jax.experimental.pallas import tpu_sc as plsc

N = 10000
E = 320000
NC = 2
NS = 16
NW = NC * NS
CH = 128
CHUNKS = E // CH
CPW = CHUNKS // NW
EXTRA = CHUNKS - CPW * NW
NP = 10240
RPT = NP // NS
BN = 2000

_MESH = plsc.VectorSubcoreMesh(core_axis_name="c", subcore_axis_name="s")
_f32 = jnp.float32


def _make_edge_scatter(w):
  @functools.partial(
      pl.kernel,
      out_type=jax.ShapeDtypeStruct((NC * NP, w), _f32),
      mesh=_MESH,
      scratch_types=[
          pltpu.VMEM((CH,), jnp.int32),
          pltpu.VMEM((CH,), jnp.int32),
          pltpu.VMEM((CH, w), _f32),
          pltpu.VMEM_SHARED((NP, w), _f32),
          pltpu.SemaphoreType.DMA,
      ],
      compiler_params=pltpu.CompilerParams(use_tc_tiling_on_sc=False),
  )
  def k(t_hbm, src_hbm, dst_hbm, zeros_hbm, out_hbm, src_v, dst_v, rows_v,
        acc, gsem):
    cid = lax.axis_index("c")
    sid = lax.axis_index("s")
    wid = sid * NC + cid
    rbase = sid * RPT
    pltpu.sync_copy(zeros_hbm.at[pl.ds(rbase, RPT)],
                    acc.at[pl.ds(rbase, RPT)])
    plsc.subcore_barrier()

    def do_chunk(c):
      ebase = c * CH
      pltpu.sync_copy(src_hbm.at[pl.ds(ebase, CH)], src_v)
      pltpu.sync_copy(dst_hbm.at[pl.ds(ebase, CH)], dst_v)
      pltpu.async_copy(t_hbm.at[src_v], rows_v, gsem).wait()
      pltpu.sync_copy(rows_v, acc.at[dst_v], add=True)

    def body(i, carry):
      do_chunk(wid * CPW + i)
      return carry

    lax.fori_loop(0, CPW, body, 0)

    @pl.when(wid < EXTRA)
    def _():
      do_chunk(NW * CPW + wid)

    plsc.subcore_barrier()
    pltpu.sync_copy(acc.at[pl.ds(rbase, RPT)],
                    out_hbm.at[pl.ds(cid * NP + rbase, RPT)])

  return k


_scat16 = _make_edge_scatter(16)
_scat32 = _make_edge_scatter(32)


@functools.partial(
    pl.kernel,
    out_type=jax.ShapeDtypeStruct((NC * NP, 16), _f32),
    mesh=_MESH,
    scratch_types=[
        pltpu.VMEM((CH,), jnp.int32),
        pltpu.VMEM((CH, 16), _f32),
        pltpu.VMEM_SHARED((NP, 16), _f32),
    ],
    compiler_params=pltpu.CompilerParams(use_tc_tiling_on_sc=False),
)
def _deg_kernel(dst_hbm, ones_hbm, zeros_hbm, out_hbm, dst_v, ones_v, acc):
  cid = lax.axis_index("c")
  sid = lax.axis_index("s")
  wid = sid * NC + cid
  rbase = sid * RPT
  pltpu.sync_copy(ones_hbm, ones_v)
  pltpu.sync_copy(zeros_hbm.at[pl.ds(rbase, RPT)], acc.at[pl.ds(rbase, RPT)])
  plsc.subcore_barrier()

  def do_chunk(c):
    pltpu.sync_copy(dst_hbm.at[pl.ds(c * CH, CH)], dst_v)
    pltpu.sync_copy(ones_v, acc.at[dst_v], add=True)

  def body(i, carry):
    do_chunk(wid * CPW + i)
    return carry

  lax.fori_loop(0, CPW, body, 0)

  @pl.when(wid < EXTRA)
  def _():
    do_chunk(NW * CPW + wid)

  plsc.subcore_barrier()
  pltpu.sync_copy(acc.at[pl.ds(rbase, RPT)],
                  out_hbm.at[pl.ds(cid * NP + rbase, RPT)])


def _rows_spec(k):
  return pl.BlockSpec((BN, k), lambda i: (i, 0))


def _full_spec(shape):
  return pl.BlockSpec(shape, lambda i: (0,) * len(shape))


def _dot(a, b):
  return jnp.dot(a, b, preferred_element_type=_f32,
                 precision=jax.lax.Precision.HIGHEST)


def _tc_call(body, ins, widths_in, out_widths):
  in_specs = []
  for a, kind in zip(ins, widths_in):
    in_specs.append(_rows_spec(a.shape[1]) if kind == "r" else
                    _full_spec(a.shape))
  out_specs = [_rows_spec(w) for w in out_widths]
  out_shape = [jax.ShapeDtypeStruct((N, w), _f32) for w in out_widths]
  res = pl.pallas_call(
      body,
      grid=(N // BN,),
      in_specs=in_specs,
      out_specs=out_specs,
      out_shape=out_shape,
  )(*ins)
  return res


def _tca(pa_ref, pb_ref, x_ref, w1_ref, u_o, t1_o):
  deg = 1.0 + pa_ref[...][:, 0:1] + pb_ref[...][:, 0:1]
  u = lax.rsqrt(deg)
  u_o[...] = u
  t1_o[...] = u * _dot(x_ref[...], w1_ref[...])


def _tcb(u_ref, pa_ref, pb_ref, t1_ref, w2_ref, b1_ref, x1_o, t2_o):
  u = u_ref[...]
  x1 = jnp.tanh(u * (pa_ref[...] + pb_ref[...] + t1_ref[...]) + b1_ref[...])
  x1_o[...] = x1
  t2_o[...] = u * _dot(x1, w2_ref[...])


def _tcc(u_ref, pa_ref, pb_ref, t2_ref, b2_ref, x2_o, t3_o):
  u = u_ref[...]
  x2 = jnp.tanh(u * (pa_ref[...] + pb_ref[...] + t2_ref[...]) + b2_ref[...])
  x2_o[...] = x2
  t3_o[...] = u * x2


def _tcd(u_ref, pa_ref, pb_ref, t3_ref, w3_ref, b3_ref, w4_ref,
         z_o, pred_o, t4_o):
  u = u_ref[...]
  m = u * (pa_ref[...] + pb_ref[...] + t3_ref[...])
  z = jnp.tanh(_dot(m, w3_ref[...]) + b3_ref[...])
  z_o[...] = z
  zmax = jnp.max(z, axis=1, keepdims=True)
  ez = jnp.exp(z - zmax)
  pred_o[...] = ez / jnp.sum(ez, axis=1, keepdims=True)
  t4_o[...] = u * _dot(z, w4_ref[...])


def _tce(u_ref, pa_ref, pb_ref, t4_ref, b4_ref, x2_ref, t5_o):
  u = u_ref[...]
  z2 = jnp.tanh(u * (pa_ref[...] + pb_ref[...] + t4_ref[...]) + b4_ref[...])
  z2 = z2 + x2_ref[...]
  t5_o[...] = u * z2


def _tcf(u_ref, pa_ref, pb_ref, t5_ref, w5_ref, b5_ref, x1_ref, t6_o):
  u = u_ref[...]
  m = u * (pa_ref[...] + pb_ref[...] + t5_ref[...])
  z1 = jnp.tanh(_dot(m, w5_ref[...]) + b5_ref[...]) + x1_ref[...]
  t6_o[...] = u * z1


def _tcg(u_ref, pa_ref, pb_ref, t6_ref, w6_ref, b6_ref, feat_o):
  u = u_ref[...]
  m = u * (pa_ref[...] + pb_ref[...] + t6_ref[...])
  feat_o[...] = jax.nn.sigmoid(_dot(m, w6_ref[...]) + b6_ref[...])


def kernel(x, edge_index, W1, b1, W2, b2, W3, b3, W4, b4, W5, b5, W6, b6):
  src = edge_index[0]
  dst = edge_index[1]
  zeros16 = jnp.zeros((NP, 16), _f32)
  zeros32 = jnp.zeros((NP, 32), _f32)
  ones = jnp.ones((CH, 16), _f32)

  degp = _deg_kernel(dst, ones, zeros16)
  dpa, dpb = degp[:N], degp[NP:NP + N]

  u, t1 = _tc_call(_tca, [dpa, dpb, x, W1], "rrrf", [1, 32])

  g1 = _scat32(t1, src, dst, zeros32)
  x1, t2 = _tc_call(_tcb, [u, g1[:N], g1[NP:NP + N], t1, W2, b1.reshape(1, -1)],
                    "rrrrff", [32, 16])

  g2 = _scat16(t2, src, dst, zeros16)
  x2, t3 = _tc_call(_tcc, [u, g2[:N], g2[NP:NP + N], t2, b2.reshape(1, -1)],
                    "rrrrf", [16, 16])

  g3 = _scat16(t3, src, dst, zeros16)
  z, pred, t4 = _tc_call(
      _tcd, [u, g3[:N], g3[NP:NP + N], t3, W3, b3.reshape(1, -1), W4],
      "rrrrfff", [40, 40, 16])

  g4 = _scat16(t4, src, dst, zeros16)
  t5, = _tc_call(_tce, [u, g4[:N], g4[NP:NP + N], t4, b4.reshape(1, -1), x2],
                 "rrrrfr", [16])

  g5 = _scat16(t5, src, dst, zeros16)
  t6, = _tc_call(_tcf, [u, g5[:N], g5[NP:NP + N], t5, W5, b5.reshape(1, -1), x1],
                 "rrrrffr", [32])

  g6 = _scat32(t6, src, dst, zeros32)
  feat, = _tc_call(_tcg, [u, g6[:N], g6[NP:NP + N], t6, W6, b6.reshape(1, -1)],
                   "rrrrff", [128])

  return (feat, z, pred)


# speedup vs baseline: 37.7444x; 2.4497x over previous
import functools

import jax
import jax.numpy as jnp
from jax import lax
from jax.experimental import pallas as pl
from jax.experimental.pallas import tpu as pltpu
from jax.experimental.pallas import tpu_sc as plsc

N = 10000
E = 320000
NC = 2
NS = 16
NW = NC * NS
CH = 128
CHUNKS = E // CH
CPW = CHUNKS // NW
EXTRA = CHUNKS - CPW * NW
NP = 10240
RPT = NP // NS
BN = 2000

_MESH = plsc.VectorSubcoreMesh(core_axis_name="c", subcore_axis_name="s")
_f32 = jnp.float32


def _make_edge_scatter(w, k):
  rounds = CPW // k
  assert k * rounds == CPW

  @functools.partial(
      pl.kernel,
      out_type=jax.ShapeDtypeStruct((NC * NP, w), _f32),
      mesh=_MESH,
      scratch_types=[
          pltpu.VMEM((CPW + 1, CH), jnp.int32),
          pltpu.VMEM((CPW + 1, CH), jnp.int32),
          pltpu.VMEM((k * CH, w), _f32),
          pltpu.VMEM_SHARED((NP, w), _f32),
          pltpu.SemaphoreType.DMA,
          pltpu.SemaphoreType.DMA,
      ],
      compiler_params=pltpu.CompilerParams(use_tc_tiling_on_sc=False),
  )
  def kfn(t_hbm, src_hbm, dst_hbm, zeros_hbm, out_hbm, src_v, dst_v, rows_v,
          acc, gsem, ssem):
    cid = lax.axis_index("c")
    sid = lax.axis_index("s")
    wid = sid * NC + cid
    rbase = sid * RPT
    pltpu.sync_copy(zeros_hbm.at[pl.ds(rbase, RPT)],
                    acc.at[pl.ds(rbase, RPT)])
    cbase = wid * CPW
    pltpu.sync_copy(src_hbm.at[pl.ds(cbase, CPW)], src_v.at[pl.ds(0, CPW)])
    pltpu.sync_copy(dst_hbm.at[pl.ds(cbase, CPW)], dst_v.at[pl.ds(0, CPW)])
    plsc.subcore_barrier()

    for r in range(rounds):
      base = r * k

      def gfire(j, carry, base=base):
        c = base + j
        pltpu.async_copy(t_hbm.at[src_v.at[c]],
                         rows_v.at[pl.ds(j * CH, CH)], gsem)
        return carry

      def gdrain(j, carry, base=base):
        c = base + j
        pltpu.make_async_copy(t_hbm.at[src_v.at[c]],
                              rows_v.at[pl.ds(j * CH, CH)], gsem).wait()
        return carry

      def sfire(j, carry, base=base):
        c = base + j
        pltpu.async_copy(rows_v.at[pl.ds(j * CH, CH)],
                         acc.at[dst_v.at[c]], ssem, add=True)
        return carry

      def sdrain(j, carry, base=base):
        c = base + j
        pltpu.make_async_copy(rows_v.at[pl.ds(j * CH, CH)],
                              acc.at[dst_v.at[c]], ssem).wait()
        return carry

      lax.fori_loop(0, k, gfire, 0)
      lax.fori_loop(0, k, gdrain, 0)
      lax.fori_loop(0, k, sfire, 0)
      lax.fori_loop(0, k, sdrain, 0)

    @pl.when(wid < EXTRA)
    def _():
      pltpu.sync_copy(src_hbm.at[pl.ds(NW * CPW + wid, 1)],
                      src_v.at[pl.ds(CPW, 1)])
      pltpu.sync_copy(dst_hbm.at[pl.ds(NW * CPW + wid, 1)],
                      dst_v.at[pl.ds(CPW, 1)])
      pltpu.async_copy(t_hbm.at[src_v.at[CPW]],
                       rows_v.at[pl.ds(0, CH)], gsem).wait()
      pltpu.sync_copy(rows_v.at[pl.ds(0, CH)], acc.at[dst_v.at[CPW]],
                      add=True)

    plsc.subcore_barrier()
    pltpu.sync_copy(acc.at[pl.ds(rbase, RPT)],
                    out_hbm.at[pl.ds(cid * NP + rbase, RPT)])

  return kfn


_scat16 = _make_edge_scatter(16, 26)
_scat32 = _make_edge_scatter(32, 13)


@functools.partial(
    pl.kernel,
    out_type=jax.ShapeDtypeStruct((NC * NP, 16), _f32),
    mesh=_MESH,
    scratch_types=[
        pltpu.VMEM((CPW + 1, CH), jnp.int32),
        pltpu.VMEM((CH, 16), _f32),
        pltpu.VMEM_SHARED((NP, 16), _f32),
        pltpu.SemaphoreType.DMA,
    ],
    compiler_params=pltpu.CompilerParams(use_tc_tiling_on_sc=False),
)
def _deg_kernel(dst_hbm, ones_hbm, zeros_hbm, out_hbm, dst_v, ones_v, acc,
                ssem):
  cid = lax.axis_index("c")
  sid = lax.axis_index("s")
  wid = sid * NC + cid
  rbase = sid * RPT
  pltpu.sync_copy(ones_hbm, ones_v)
  pltpu.sync_copy(zeros_hbm.at[pl.ds(rbase, RPT)], acc.at[pl.ds(rbase, RPT)])
  pltpu.sync_copy(dst_hbm.at[pl.ds(wid * CPW, CPW)], dst_v.at[pl.ds(0, CPW)])
  plsc.subcore_barrier()

  def sfire(j, carry):
    pltpu.async_copy(ones_v, acc.at[dst_v.at[j]], ssem, add=True)
    return carry

  def sdrain(j, carry):
    pltpu.make_async_copy(ones_v, acc.at[dst_v.at[j]], ssem).wait()
    return carry

  lax.fori_loop(0, CPW, sfire, 0)
  lax.fori_loop(0, CPW, sdrain, 0)

  @pl.when(wid < EXTRA)
  def _():
    pltpu.sync_copy(dst_hbm.at[pl.ds(NW * CPW + wid, 1)],
                    dst_v.at[pl.ds(CPW, 1)])
    pltpu.sync_copy(ones_v, acc.at[dst_v.at[CPW]], add=True)

  plsc.subcore_barrier()
  pltpu.sync_copy(acc.at[pl.ds(rbase, RPT)],
                  out_hbm.at[pl.ds(cid * NP + rbase, RPT)])


def _rows_spec(k):
  return pl.BlockSpec((BN, k), lambda i: (i, 0))


def _full_spec(shape):
  return pl.BlockSpec(shape, lambda i: (0,) * len(shape))


def _dot(a, b):
  return jnp.dot(a, b, preferred_element_type=_f32,
                 precision=jax.lax.Precision.HIGHEST)


def _tc_call(body, ins, widths_in, out_widths):
  in_specs = []
  for a, kind in zip(ins, widths_in):
    in_specs.append(_rows_spec(a.shape[1]) if kind == "r" else
                    _full_spec(a.shape))
  out_specs = [_rows_spec(w) for w in out_widths]
  out_shape = [jax.ShapeDtypeStruct((N, w), _f32) for w in out_widths]
  res = pl.pallas_call(
      body,
      grid=(N // BN,),
      in_specs=in_specs,
      out_specs=out_specs,
      out_shape=out_shape,
  )(*ins)
  return res


def _tca(pa_ref, pb_ref, x_ref, w1_ref, u_o, t1_o):
  deg = 1.0 + pa_ref[...][:, 0:1] + pb_ref[...][:, 0:1]
  u = lax.rsqrt(deg)
  u_o[...] = u
  t1_o[...] = u * _dot(x_ref[...], w1_ref[...])


def _tcb(u_ref, pa_ref, pb_ref, t1_ref, w2_ref, b1_ref, x1_o, t2_o):
  u = u_ref[...]
  x1 = jnp.tanh(u * (pa_ref[...] + pb_ref[...] + t1_ref[...]) + b1_ref[...])
  x1_o[...] = x1
  t2_o[...] = u * _dot(x1, w2_ref[...])


def _tcc(u_ref, pa_ref, pb_ref, t2_ref, b2_ref, x2_o, t3_o):
  u = u_ref[...]
  x2 = jnp.tanh(u * (pa_ref[...] + pb_ref[...] + t2_ref[...]) + b2_ref[...])
  x2_o[...] = x2
  t3_o[...] = u * x2


def _tcd(u_ref, pa_ref, pb_ref, t3_ref, w3_ref, b3_ref, w4_ref,
         z_o, pred_o, t4_o):
  u = u_ref[...]
  m = u * (pa_ref[...] + pb_ref[...] + t3_ref[...])
  z = jnp.tanh(_dot(m, w3_ref[...]) + b3_ref[...])
  z_o[...] = z
  zmax = jnp.max(z, axis=1, keepdims=True)
  ez = jnp.exp(z - zmax)
  pred_o[...] = ez / jnp.sum(ez, axis=1, keepdims=True)
  t4_o[...] = u * _dot(z, w4_ref[...])


def _tce(u_ref, pa_ref, pb_ref, t4_ref, b4_ref, x2_ref, t5_o):
  u = u_ref[...]
  z2 = jnp.tanh(u * (pa_ref[...] + pb_ref[...] + t4_ref[...]) + b4_ref[...])
  z2 = z2 + x2_ref[...]
  t5_o[...] = u * z2


def _tcf(u_ref, pa_ref, pb_ref, t5_ref, w5_ref, b5_ref, x1_ref, t6_o):
  u = u_ref[...]
  m = u * (pa_ref[...] + pb_ref[...] + t5_ref[...])
  z1 = jnp.tanh(_dot(m, w5_ref[...]) + b5_ref[...]) + x1_ref[...]
  t6_o[...] = u * z1


def _tcg(u_ref, pa_ref, pb_ref, t6_ref, w6_ref, b6_ref, feat_o):
  u = u_ref[...]
  m = u * (pa_ref[...] + pb_ref[...] + t6_ref[...])
  feat_o[...] = jax.nn.sigmoid(_dot(m, w6_ref[...]) + b6_ref[...])


def kernel(x, edge_index, W1, b1, W2, b2, W3, b3, W4, b4, W5, b5, W6, b6):
  src = edge_index[0].reshape(CHUNKS, CH)
  dst = edge_index[1].reshape(CHUNKS, CH)
  zeros16 = jnp.zeros((NP, 16), _f32)
  zeros32 = jnp.zeros((NP, 32), _f32)
  ones = jnp.ones((CH, 16), _f32)

  degp = _deg_kernel(dst, ones, zeros16)
  dpa, dpb = degp[:N], degp[NP:NP + N]

  u, t1 = _tc_call(_tca, [dpa, dpb, x, W1], "rrrf", [1, 32])

  g1 = _scat32(t1, src, dst, zeros32)
  x1, t2 = _tc_call(_tcb, [u, g1[:N], g1[NP:NP + N], t1, W2, b1.reshape(1, -1)],
                    "rrrrff", [32, 16])

  g2 = _scat16(t2, src, dst, zeros16)
  x2, t3 = _tc_call(_tcc, [u, g2[:N], g2[NP:NP + N], t2, b2.reshape(1, -1)],
                    "rrrrf", [16, 16])

  g3 = _scat16(t3, src, dst, zeros16)
  z, pred, t4 = _tc_call(
      _tcd, [u, g3[:N], g3[NP:NP + N], t3, W3, b3.reshape(1, -1), W4],
      "rrrrfff", [40, 40, 16])

  g4 = _scat16(t4, src, dst, zeros16)
  t5, = _tc_call(_tce, [u, g4[:N], g4[NP:NP + N], t4, b4.reshape(1, -1), x2],
                 "rrrrfr", [16])

  g5 = _scat16(t5, src, dst, zeros16)
  t6, = _tc_call(_tcf, [u, g5[:N], g5[NP:NP + N], t5, W5, b5.reshape(1, -1), x1],
                 "rrrrffr", [32])

  g6 = _scat32(t6, src, dst, zeros32)
  feat, = _tc_call(_tcg, [u, g6[:N], g6[NP:NP + N], t6, W6, b6.reshape(1, -1)],
                   "rrrrff", [128])

  return (feat, z, pred)


# speedup vs baseline: 41.3082x; 1.0944x over previous
import functools

import jax
import jax.numpy as jnp
from jax import lax
from jax.experimental import pallas as pl
from jax.experimental.pallas import tpu as pltpu
from jax.experimental.pallas import tpu_sc as plsc

N = 10000
E = 320000
NC = 2
NS = 16
NW = NC * NS
CH = 128
CHUNKS = E // CH
CPW = CHUNKS // NW
EXTRA = CHUNKS - CPW * NW
NP = 10240
RPT = NP // NS
BN = 1280

_MESH = plsc.VectorSubcoreMesh(core_axis_name="c", subcore_axis_name="s")
_f32 = jnp.float32


def _make_edge_scatter(w, k):
  rounds = CPW // k
  assert k * rounds == CPW

  @functools.partial(
      pl.kernel,
      out_type=jax.ShapeDtypeStruct((NC * NP, w), _f32),
      mesh=_MESH,
      scratch_types=[
          pltpu.VMEM((CPW + 1, CH), jnp.int32),
          pltpu.VMEM((CPW + 1, CH), jnp.int32),
          pltpu.VMEM((k * CH, w), _f32),
          pltpu.VMEM_SHARED((NP, w), _f32),
          pltpu.SemaphoreType.DMA,
          pltpu.SemaphoreType.DMA,
      ],
      compiler_params=pltpu.CompilerParams(use_tc_tiling_on_sc=False),
  )
  def kfn(t_hbm, src_hbm, dst_hbm, zeros_hbm, out_hbm, src_v, dst_v, rows_v,
          acc, gsem, ssem):
    cid = lax.axis_index("c")
    sid = lax.axis_index("s")
    wid = sid * NC + cid
    rbase = sid * RPT
    pltpu.sync_copy(zeros_hbm.at[pl.ds(rbase, RPT)],
                    acc.at[pl.ds(rbase, RPT)])
    cbase = wid * CPW
    pltpu.sync_copy(src_hbm.at[pl.ds(cbase, CPW)], src_v.at[pl.ds(0, CPW)])
    pltpu.sync_copy(dst_hbm.at[pl.ds(cbase, CPW)], dst_v.at[pl.ds(0, CPW)])
    plsc.subcore_barrier()

    for r in range(rounds):
      base = r * k

      def gfire(j, carry, base=base):
        c = base + j
        pltpu.async_copy(t_hbm.at[src_v.at[c]],
                         rows_v.at[pl.ds(j * CH, CH)], gsem)
        return carry

      def gdrain(j, carry, base=base):
        c = base + j
        pltpu.make_async_copy(t_hbm.at[src_v.at[c]],
                              rows_v.at[pl.ds(j * CH, CH)], gsem).wait()
        return carry

      def sfire(j, carry, base=base):
        c = base + j
        pltpu.async_copy(rows_v.at[pl.ds(j * CH, CH)],
                         acc.at[dst_v.at[c]], ssem, add=True)
        return carry

      def sdrain(j, carry, base=base):
        c = base + j
        pltpu.make_async_copy(rows_v.at[pl.ds(j * CH, CH)],
                              acc.at[dst_v.at[c]], ssem).wait()
        return carry

      lax.fori_loop(0, k, gfire, 0)
      lax.fori_loop(0, k, gdrain, 0)
      lax.fori_loop(0, k, sfire, 0)
      lax.fori_loop(0, k, sdrain, 0)

    @pl.when(wid < EXTRA)
    def _():
      pltpu.sync_copy(src_hbm.at[pl.ds(NW * CPW + wid, 1)],
                      src_v.at[pl.ds(CPW, 1)])
      pltpu.sync_copy(dst_hbm.at[pl.ds(NW * CPW + wid, 1)],
                      dst_v.at[pl.ds(CPW, 1)])
      pltpu.async_copy(t_hbm.at[src_v.at[CPW]],
                       rows_v.at[pl.ds(0, CH)], gsem).wait()
      pltpu.sync_copy(rows_v.at[pl.ds(0, CH)], acc.at[dst_v.at[CPW]],
                      add=True)

    plsc.subcore_barrier()
    pltpu.sync_copy(acc.at[pl.ds(rbase, RPT)],
                    out_hbm.at[pl.ds(cid * NP + rbase, RPT)])

  return kfn


_scat16 = _make_edge_scatter(16, 26)
_scat32 = _make_edge_scatter(32, 13)


@functools.partial(
    pl.kernel,
    out_type=jax.ShapeDtypeStruct((NC * NP, 16), _f32),
    mesh=_MESH,
    scratch_types=[
        pltpu.VMEM((CPW + 1, CH), jnp.int32),
        pltpu.VMEM((CH, 16), _f32),
        pltpu.VMEM_SHARED((NP, 16), _f32),
        pltpu.SemaphoreType.DMA,
    ],
    compiler_params=pltpu.CompilerParams(use_tc_tiling_on_sc=False),
)
def _deg_kernel(dst_hbm, ones_hbm, zeros_hbm, out_hbm, dst_v, ones_v, acc,
                ssem):
  cid = lax.axis_index("c")
  sid = lax.axis_index("s")
  wid = sid * NC + cid
  rbase = sid * RPT
  pltpu.sync_copy(ones_hbm, ones_v)
  pltpu.sync_copy(zeros_hbm.at[pl.ds(rbase, RPT)], acc.at[pl.ds(rbase, RPT)])
  pltpu.sync_copy(dst_hbm.at[pl.ds(wid * CPW, CPW)], dst_v.at[pl.ds(0, CPW)])
  plsc.subcore_barrier()

  def sfire(j, carry):
    pltpu.async_copy(ones_v, acc.at[dst_v.at[j]], ssem, add=True)
    return carry

  def sdrain(j, carry):
    pltpu.make_async_copy(ones_v, acc.at[dst_v.at[j]], ssem).wait()
    return carry

  lax.fori_loop(0, CPW, sfire, 0)
  lax.fori_loop(0, CPW, sdrain, 0)

  @pl.when(wid < EXTRA)
  def _():
    pltpu.sync_copy(dst_hbm.at[pl.ds(NW * CPW + wid, 1)],
                    dst_v.at[pl.ds(CPW, 1)])
    pltpu.sync_copy(ones_v, acc.at[dst_v.at[CPW]], add=True)

  plsc.subcore_barrier()
  pltpu.sync_copy(acc.at[pl.ds(rbase, RPT)],
                  out_hbm.at[pl.ds(cid * NP + rbase, RPT)])


def _rows_spec(k):
  return pl.BlockSpec((BN, k), lambda i: (i, 0))


def _full_spec(shape):
  return pl.BlockSpec(shape, lambda i: (0,) * len(shape))


def _dot(a, b):
  return jnp.dot(a, b, preferred_element_type=_f32,
                 precision=jax.lax.Precision.HIGHEST)


def _tc_call(body, ins, widths_in, out_widths):
  in_specs = []
  for a, kind in zip(ins, widths_in):
    if kind == "r":
      in_specs.append(_rows_spec(a.shape[1]))
    elif kind == "a":
      in_specs.append(pl.BlockSpec((BN, a.shape[1]), lambda i: (i, 0)))
    elif kind == "b":
      in_specs.append(pl.BlockSpec((BN, a.shape[1]),
                                   lambda i: (i + NP // BN, 0)))
    else:
      in_specs.append(_full_spec(a.shape))
  out_specs = [_rows_spec(w) for w in out_widths]
  out_shape = [jax.ShapeDtypeStruct((N, w), _f32) for w in out_widths]
  res = pl.pallas_call(
      body,
      grid=((N + BN - 1) // BN,),
      in_specs=in_specs,
      out_specs=out_specs,
      out_shape=out_shape,
  )(*ins)
  return res


def _tca1(x_ref, w1_ref, h1_o):
  h1_o[...] = _dot(x_ref[...], w1_ref[...])


def _tca2(pa_ref, pb_ref, h1_ref, u_o, t1_o):
  deg = 1.0 + pa_ref[...][:, 0:1] + pb_ref[...][:, 0:1]
  u = lax.rsqrt(deg)
  u_o[...] = u
  t1_o[...] = u * h1_ref[...]


def _tcb(u_ref, pa_ref, pb_ref, t1_ref, w2_ref, b1_ref, x1_o, t2_o):
  u = u_ref[...]
  x1 = jnp.tanh(u * (pa_ref[...] + pb_ref[...] + t1_ref[...]) + b1_ref[...])
  x1_o[...] = x1
  t2_o[...] = u * _dot(x1, w2_ref[...])


def _tcc(u_ref, pa_ref, pb_ref, t2_ref, b2_ref, x2_o, t3_o):
  u = u_ref[...]
  x2 = jnp.tanh(u * (pa_ref[...] + pb_ref[...] + t2_ref[...]) + b2_ref[...])
  x2_o[...] = x2
  t3_o[...] = u * x2


def _tcd(u_ref, pa_ref, pb_ref, t3_ref, w3_ref, b3_ref, w4_ref,
         z_o, pred_o, t4_o):
  u = u_ref[...]
  m = u * (pa_ref[...] + pb_ref[...] + t3_ref[...])
  z = jnp.tanh(_dot(m, w3_ref[...]) + b3_ref[...])
  z_o[...] = z
  zmax = jnp.max(z, axis=1, keepdims=True)
  ez = jnp.exp(z - zmax)
  pred_o[...] = ez / jnp.sum(ez, axis=1, keepdims=True)
  t4_o[...] = u * _dot(z, w4_ref[...])


def _tce(u_ref, pa_ref, pb_ref, t4_ref, b4_ref, x2_ref, t5_o):
  u = u_ref[...]
  z2 = jnp.tanh(u * (pa_ref[...] + pb_ref[...] + t4_ref[...]) + b4_ref[...])
  z2 = z2 + x2_ref[...]
  t5_o[...] = u * z2


def _tcf(u_ref, pa_ref, pb_ref, t5_ref, w5_ref, b5_ref, x1_ref, t6_o):
  u = u_ref[...]
  m = u * (pa_ref[...] + pb_ref[...] + t5_ref[...])
  z1 = jnp.tanh(_dot(m, w5_ref[...]) + b5_ref[...]) + x1_ref[...]
  t6_o[...] = u * z1


def _tcg(u_ref, pa_ref, pb_ref, t6_ref, w6_ref, b6_ref, feat_o):
  u = u_ref[...]
  m = u * (pa_ref[...] + pb_ref[...] + t6_ref[...])
  feat_o[...] = jax.nn.sigmoid(_dot(m, w6_ref[...]) + b6_ref[...])


def kernel(x, edge_index, W1, b1, W2, b2, W3, b3, W4, b4, W5, b5, W6, b6):
  src = edge_index[0].reshape(CHUNKS, CH)
  dst = edge_index[1].reshape(CHUNKS, CH)
  zeros16 = jnp.zeros((NP, 16), _f32)
  zeros32 = jnp.zeros((NP, 32), _f32)
  ones = jnp.ones((CH, 16), _f32)

  degp = _deg_kernel(dst, ones, zeros16)
  h1, = _tc_call(_tca1, [x, W1], "rf", [32])

  u, t1 = _tc_call(_tca2, [degp, degp, h1], "abr", [1, 32])

  g1 = _scat32(t1, src, dst, zeros32)
  x1, t2 = _tc_call(_tcb, [u, g1, g1, t1, W2, b1.reshape(1, -1)],
                    "rabrff", [32, 16])

  g2 = _scat16(t2, src, dst, zeros16)
  x2, t3 = _tc_call(_tcc, [u, g2, g2, t2, b2.reshape(1, -1)],
                    "rabrf", [16, 16])

  g3 = _scat16(t3, src, dst, zeros16)
  z, pred, t4 = _tc_call(
      _tcd, [u, g3, g3, t3, W3, b3.reshape(1, -1), W4],
      "rabrfff", [40, 40, 16])

  g4 = _scat16(t4, src, dst, zeros16)
  t5, = _tc_call(_tce, [u, g4, g4, t4, b4.reshape(1, -1), x2],
                 "rabrfr", [16])

  g5 = _scat16(t5, src, dst, zeros16)
  t6, = _tc_call(_tcf, [u, g5, g5, t5, W5, b5.reshape(1, -1), x1],
                 "rabrffr", [32])

  g6 = _scat32(t6, src, dst, zeros32)
  feat, = _tc_call(_tcg, [u, g6, g6, t6, W6, b6.reshape(1, -1)],
                   "rabrff", [128])

  return (feat, z, pred)


# speedup vs baseline: 44.1526x; 1.0689x over previous
import functools

import jax
import jax.numpy as jnp
from jax import lax
from jax.experimental import pallas as pl
from jax.experimental.pallas import tpu as pltpu
from jax.experimental.pallas import tpu_sc as plsc

N = 10000
E = 320000
NC = 2
NS = 16
NW = NC * NS
CH = 128
CHUNKS = E // CH
CPW = CHUNKS // NW
EXTRA = CHUNKS - CPW * NW
NP = 10240
RPT = NP // NS
BN = 1280

_MESH = plsc.VectorSubcoreMesh(core_axis_name="c", subcore_axis_name="s")
_f32 = jnp.float32


def _make_edge_scatter(w, k):
  rounds = CPW // k
  assert k * rounds == CPW

  @functools.partial(
      pl.kernel,
      out_type=jax.ShapeDtypeStruct((NC * NP, w), _f32),
      mesh=_MESH,
      scratch_types=[
          pltpu.VMEM((CPW + 1, CH), jnp.int32),
          pltpu.VMEM((CPW + 1, CH), jnp.int32),
          pltpu.VMEM((2 * k * CH, w), _f32),
          pltpu.VMEM_SHARED((NP, w), _f32),
          pltpu.SemaphoreType.DMA,
          pltpu.SemaphoreType.DMA,
      ],
      compiler_params=pltpu.CompilerParams(use_tc_tiling_on_sc=False),
  )
  def kfn(t_hbm, src_hbm, dst_hbm, zeros_hbm, out_hbm, src_v, dst_v, rows_v,
          acc, gsem, ssem):
    cid = lax.axis_index("c")
    sid = lax.axis_index("s")
    wid = sid * NC + cid
    rbase = sid * RPT
    pltpu.sync_copy(zeros_hbm.at[pl.ds(rbase, RPT)],
                    acc.at[pl.ds(rbase, RPT)])
    cbase = wid * CPW
    pltpu.sync_copy(src_hbm.at[pl.ds(cbase, CPW)], src_v.at[pl.ds(0, CPW)])
    pltpu.sync_copy(dst_hbm.at[pl.ds(cbase, CPW)], dst_v.at[pl.ds(0, CPW)])
    plsc.subcore_barrier()

    def gfire(j, carry, base=0, boff=0):
      pltpu.async_copy(t_hbm.at[src_v.at[base + j]],
                       rows_v.at[pl.ds((boff + j) * CH, CH)], gsem)
      return carry

    def gdrain(j, carry, base=0, boff=0):
      pltpu.make_async_copy(t_hbm.at[src_v.at[base + j]],
                            rows_v.at[pl.ds((boff + j) * CH, CH)],
                            gsem).wait()
      return carry

    def sfire(j, carry, base=0, boff=0):
      pltpu.async_copy(rows_v.at[pl.ds((boff + j) * CH, CH)],
                       acc.at[dst_v.at[base + j]], ssem, add=True)
      return carry

    def sdrain(j, carry, base=0, boff=0):
      pltpu.make_async_copy(rows_v.at[pl.ds((boff + j) * CH, CH)],
                            acc.at[dst_v.at[base + j]], ssem).wait()
      return carry

    for r in range(rounds):
      base, boff = r * k, (r % 2) * k
      if r >= 2:
        pb2, bo2 = (r - 2) * k, (r % 2) * k
        lax.fori_loop(0, k, functools.partial(sdrain, base=pb2, boff=bo2), 0)
      lax.fori_loop(0, k, functools.partial(gfire, base=base, boff=boff), 0)
      lax.fori_loop(0, k, functools.partial(gdrain, base=base, boff=boff), 0)
      lax.fori_loop(0, k, functools.partial(sfire, base=base, boff=boff), 0)
    for r in (max(rounds - 2, 0), rounds - 1):
      base, boff = r * k, (r % 2) * k
      lax.fori_loop(0, k, functools.partial(sdrain, base=base, boff=boff), 0)

    @pl.when(wid < EXTRA)
    def _():
      pltpu.sync_copy(src_hbm.at[pl.ds(NW * CPW + wid, 1)],
                      src_v.at[pl.ds(CPW, 1)])
      pltpu.sync_copy(dst_hbm.at[pl.ds(NW * CPW + wid, 1)],
                      dst_v.at[pl.ds(CPW, 1)])
      pltpu.async_copy(t_hbm.at[src_v.at[CPW]],
                       rows_v.at[pl.ds(0, CH)], gsem).wait()
      pltpu.sync_copy(rows_v.at[pl.ds(0, CH)], acc.at[dst_v.at[CPW]],
                      add=True)

    plsc.subcore_barrier()
    pltpu.sync_copy(acc.at[pl.ds(rbase, RPT)],
                    out_hbm.at[pl.ds(cid * NP + rbase, RPT)])

  return kfn


_scat16 = _make_edge_scatter(16, 13)
_scat32 = _make_edge_scatter(32, 6)


@functools.partial(
    pl.kernel,
    out_type=jax.ShapeDtypeStruct((NC * NP, 16), _f32),
    mesh=_MESH,
    scratch_types=[
        pltpu.VMEM((CPW + 1, CH), jnp.int32),
        pltpu.VMEM((CH, 16), _f32),
        pltpu.VMEM_SHARED((NP, 16), _f32),
        pltpu.SemaphoreType.DMA,
    ],
    compiler_params=pltpu.CompilerParams(use_tc_tiling_on_sc=False),
)
def _deg_kernel(dst_hbm, ones_hbm, zeros_hbm, out_hbm, dst_v, ones_v, acc,
                ssem):
  cid = lax.axis_index("c")
  sid = lax.axis_index("s")
  wid = sid * NC + cid
  rbase = sid * RPT
  pltpu.sync_copy(ones_hbm, ones_v)
  pltpu.sync_copy(zeros_hbm.at[pl.ds(rbase, RPT)], acc.at[pl.ds(rbase, RPT)])
  pltpu.sync_copy(dst_hbm.at[pl.ds(wid * CPW, CPW)], dst_v.at[pl.ds(0, CPW)])
  plsc.subcore_barrier()

  def sfire(j, carry):
    pltpu.async_copy(ones_v, acc.at[dst_v.at[j]], ssem, add=True)
    return carry

  def sdrain(j, carry):
    pltpu.make_async_copy(ones_v, acc.at[dst_v.at[j]], ssem).wait()
    return carry

  lax.fori_loop(0, CPW, sfire, 0)
  lax.fori_loop(0, CPW, sdrain, 0)

  @pl.when(wid < EXTRA)
  def _():
    pltpu.sync_copy(dst_hbm.at[pl.ds(NW * CPW + wid, 1)],
                    dst_v.at[pl.ds(CPW, 1)])
    pltpu.sync_copy(ones_v, acc.at[dst_v.at[CPW]], add=True)

  plsc.subcore_barrier()
  pltpu.sync_copy(acc.at[pl.ds(rbase, RPT)],
                  out_hbm.at[pl.ds(cid * NP + rbase, RPT)])


def _rows_spec(k):
  return pl.BlockSpec((BN, k), lambda i: (i, 0))


def _full_spec(shape):
  return pl.BlockSpec(shape, lambda i: (0,) * len(shape))


def _dot(a, b):
  return jnp.dot(a, b, preferred_element_type=_f32,
                 precision=jax.lax.Precision.HIGHEST)


def _tc_call(body, ins, widths_in, out_widths):
  in_specs = []
  for a, kind in zip(ins, widths_in):
    if kind == "r":
      in_specs.append(_rows_spec(a.shape[1]))
    elif kind == "a":
      in_specs.append(pl.BlockSpec((BN, a.shape[1]), lambda i: (i, 0)))
    elif kind == "b":
      in_specs.append(pl.BlockSpec((BN, a.shape[1]),
                                   lambda i: (i + NP // BN, 0)))
    else:
      in_specs.append(_full_spec(a.shape))
  out_specs = [_rows_spec(w) for w in out_widths]
  out_shape = [jax.ShapeDtypeStruct((N, w), _f32) for w in out_widths]
  res = pl.pallas_call(
      body,
      grid=((N + BN - 1) // BN,),
      in_specs=in_specs,
      out_specs=out_specs,
      out_shape=out_shape,
  )(*ins)
  return res


def _tca1(x_ref, w1_ref, h1_o):
  h1_o[...] = _dot(x_ref[...], w1_ref[...])


def _tca2(pa_ref, pb_ref, h1_ref, u_o, t1_o):
  deg = 1.0 + pa_ref[...][:, 0:1] + pb_ref[...][:, 0:1]
  u = lax.rsqrt(deg)
  u_o[...] = u
  t1_o[...] = u * h1_ref[...]


def _tcb(u_ref, pa_ref, pb_ref, t1_ref, w2_ref, b1_ref, x1_o, t2_o):
  u = u_ref[...]
  x1 = jnp.tanh(u * (pa_ref[...] + pb_ref[...] + t1_ref[...]) + b1_ref[...])
  x1_o[...] = x1
  t2_o[...] = u * _dot(x1, w2_ref[...])


def _tcc(u_ref, pa_ref, pb_ref, t2_ref, b2_ref, x2_o, t3_o):
  u = u_ref[...]
  x2 = jnp.tanh(u * (pa_ref[...] + pb_ref[...] + t2_ref[...]) + b2_ref[...])
  x2_o[...] = x2
  t3_o[...] = u * x2


def _tcd(u_ref, pa_ref, pb_ref, t3_ref, w3_ref, b3_ref, w4_ref,
         z_o, pred_o, t4_o):
  u = u_ref[...]
  m = u * (pa_ref[...] + pb_ref[...] + t3_ref[...])
  z = jnp.tanh(_dot(m, w3_ref[...]) + b3_ref[...])
  z_o[...] = z
  zmax = jnp.max(z, axis=1, keepdims=True)
  ez = jnp.exp(z - zmax)
  pred_o[...] = ez / jnp.sum(ez, axis=1, keepdims=True)
  t4_o[...] = u * _dot(z, w4_ref[...])


def _tce(u_ref, pa_ref, pb_ref, t4_ref, b4_ref, x2_ref, t5_o):
  u = u_ref[...]
  z2 = jnp.tanh(u * (pa_ref[...] + pb_ref[...] + t4_ref[...]) + b4_ref[...])
  z2 = z2 + x2_ref[...]
  t5_o[...] = u * z2


def _tcf(u_ref, pa_ref, pb_ref, t5_ref, w5_ref, b5_ref, x1_ref, t6_o):
  u = u_ref[...]
  m = u * (pa_ref[...] + pb_ref[...] + t5_ref[...])
  z1 = jnp.tanh(_dot(m, w5_ref[...]) + b5_ref[...]) + x1_ref[...]
  t6_o[...] = u * z1


def _tcg(u_ref, pa_ref, pb_ref, t6_ref, w6_ref, b6_ref, feat_o):
  u = u_ref[...]
  m = u * (pa_ref[...] + pb_ref[...] + t6_ref[...])
  feat_o[...] = jax.nn.sigmoid(_dot(m, w6_ref[...]) + b6_ref[...])


def kernel(x, edge_index, W1, b1, W2, b2, W3, b3, W4, b4, W5, b5, W6, b6):
  src = edge_index[0].reshape(CHUNKS, CH)
  dst = edge_index[1].reshape(CHUNKS, CH)
  zeros16 = jnp.zeros((NP, 16), _f32)
  zeros32 = jnp.zeros((NP, 32), _f32)
  ones = jnp.ones((CH, 16), _f32)

  degp = _deg_kernel(dst, ones, zeros16)
  h1, = _tc_call(_tca1, [x, W1], "rf", [32])

  u, t1 = _tc_call(_tca2, [degp, degp, h1], "abr", [1, 32])

  g1 = _scat32(t1, src, dst, zeros32)
  x1, t2 = _tc_call(_tcb, [u, g1, g1, t1, W2, b1.reshape(1, -1)],
                    "rabrff", [32, 16])

  g2 = _scat16(t2, src, dst, zeros16)
  x2, t3 = _tc_call(_tcc, [u, g2, g2, t2, b2.reshape(1, -1)],
                    "rabrf", [16, 16])

  g3 = _scat16(t3, src, dst, zeros16)
  z, pred, t4 = _tc_call(
      _tcd, [u, g3, g3, t3, W3, b3.reshape(1, -1), W4],
      "rabrfff", [40, 40, 16])

  g4 = _scat16(t4, src, dst, zeros16)
  t5, = _tc_call(_tce, [u, g4, g4, t4, b4.reshape(1, -1), x2],
                 "rabrfr", [16])

  g5 = _scat16(t5, src, dst, zeros16)
  t6, = _tc_call(_tcf, [u, g5, g5, t5, W5, b5.reshape(1, -1), x1],
                 "rabrffr", [32])

  g6 = _scat32(t6, src, dst, zeros32)
  feat, = _tc_call(_tcg, [u, g6, g6, t6, W6, b6.reshape(1, -1)],
                   "rabrff", [128])

  return (feat, z, pred)


# speedup vs baseline: 45.5837x; 1.0324x over previous
import functools

import jax
import jax.numpy as jnp
from jax import lax
from jax.experimental import pallas as pl
from jax.experimental.pallas import tpu as pltpu
from jax.experimental.pallas import tpu_sc as plsc

N = 10000
E = 320000
NC = 2
NS = 16
NW = NC * NS
CH = 128
CHUNKS = E // CH
CPW = CHUNKS // NW
EXTRA = CHUNKS - CPW * NW
NP = 10240
RPT = NP // NS
BN = 2560

_MESH = plsc.VectorSubcoreMesh(core_axis_name="c", subcore_axis_name="s")
_f32 = jnp.float32


def _make_edge_scatter(w, k):
  rounds = CPW // k
  assert k * rounds == CPW

  @functools.partial(
      pl.kernel,
      out_type=jax.ShapeDtypeStruct((NC * NP, w), _f32),
      mesh=_MESH,
      scratch_types=[
          pltpu.VMEM((CPW + 1, CH), jnp.int32),
          pltpu.VMEM((CPW + 1, CH), jnp.int32),
          pltpu.VMEM((2 * k * CH, w), _f32),
          pltpu.VMEM_SHARED((NP, w), _f32),
          pltpu.SemaphoreType.DMA,
          pltpu.SemaphoreType.DMA,
      ],
      compiler_params=pltpu.CompilerParams(use_tc_tiling_on_sc=False),
  )
  def kfn(t_hbm, src_hbm, dst_hbm, zeros_hbm, out_hbm, src_v, dst_v, rows_v,
          acc, gsem, ssem):
    cid = lax.axis_index("c")
    sid = lax.axis_index("s")
    wid = sid * NC + cid
    rbase = sid * RPT
    pltpu.sync_copy(zeros_hbm.at[pl.ds(rbase, RPT)],
                    acc.at[pl.ds(rbase, RPT)])
    cbase = wid * CPW
    pltpu.sync_copy(src_hbm.at[pl.ds(cbase, CPW)], src_v.at[pl.ds(0, CPW)])
    pltpu.sync_copy(dst_hbm.at[pl.ds(cbase, CPW)], dst_v.at[pl.ds(0, CPW)])
    plsc.subcore_barrier()

    def gfire(j, carry, base=0, boff=0):
      pltpu.async_copy(t_hbm.at[src_v.at[base + j]],
                       rows_v.at[pl.ds((boff + j) * CH, CH)], gsem)
      return carry

    def gdrain(j, carry, base=0, boff=0):
      pltpu.make_async_copy(t_hbm.at[src_v.at[base + j]],
                            rows_v.at[pl.ds((boff + j) * CH, CH)],
                            gsem).wait()
      return carry

    def sfire(j, carry, base=0, boff=0):
      pltpu.async_copy(rows_v.at[pl.ds((boff + j) * CH, CH)],
                       acc.at[dst_v.at[base + j]], ssem, add=True)
      return carry

    def sdrain(j, carry, base=0, boff=0):
      pltpu.make_async_copy(rows_v.at[pl.ds((boff + j) * CH, CH)],
                            acc.at[dst_v.at[base + j]], ssem).wait()
      return carry

    for r in range(rounds):
      base, boff = r * k, (r % 2) * k
      if r >= 2:
        pb2, bo2 = (r - 2) * k, (r % 2) * k
        lax.fori_loop(0, k, functools.partial(sdrain, base=pb2, boff=bo2), 0)
      lax.fori_loop(0, k, functools.partial(gfire, base=base, boff=boff), 0)
      lax.fori_loop(0, k, functools.partial(gdrain, base=base, boff=boff), 0)
      lax.fori_loop(0, k, functools.partial(sfire, base=base, boff=boff), 0)
    for r in (max(rounds - 2, 0), rounds - 1):
      base, boff = r * k, (r % 2) * k
      lax.fori_loop(0, k, functools.partial(sdrain, base=base, boff=boff), 0)

    @pl.when(wid < EXTRA)
    def _():
      pltpu.sync_copy(src_hbm.at[pl.ds(NW * CPW + wid, 1)],
                      src_v.at[pl.ds(CPW, 1)])
      pltpu.sync_copy(dst_hbm.at[pl.ds(NW * CPW + wid, 1)],
                      dst_v.at[pl.ds(CPW, 1)])
      pltpu.async_copy(t_hbm.at[src_v.at[CPW]],
                       rows_v.at[pl.ds(0, CH)], gsem).wait()
      pltpu.sync_copy(rows_v.at[pl.ds(0, CH)], acc.at[dst_v.at[CPW]],
                      add=True)

    plsc.subcore_barrier()
    pltpu.sync_copy(acc.at[pl.ds(rbase, RPT)],
                    out_hbm.at[pl.ds(cid * NP + rbase, RPT)])

  return kfn


_scat16 = _make_edge_scatter(16, 13)
_scat32 = _make_edge_scatter(32, 6)


@functools.partial(
    pl.kernel,
    out_type=jax.ShapeDtypeStruct((NC * NP, 16), _f32),
    mesh=_MESH,
    scratch_types=[
        pltpu.VMEM((CPW + 1, CH), jnp.int32),
        pltpu.VMEM((CH, 16), _f32),
        pltpu.VMEM_SHARED((NP, 16), _f32),
        pltpu.SemaphoreType.DMA,
    ],
    compiler_params=pltpu.CompilerParams(use_tc_tiling_on_sc=False),
)
def _deg_kernel(dst_hbm, ones_hbm, zeros_hbm, out_hbm, dst_v, ones_v, acc,
                ssem):
  cid = lax.axis_index("c")
  sid = lax.axis_index("s")
  wid = sid * NC + cid
  rbase = sid * RPT
  pltpu.sync_copy(ones_hbm, ones_v)
  pltpu.sync_copy(zeros_hbm.at[pl.ds(rbase, RPT)], acc.at[pl.ds(rbase, RPT)])
  pltpu.sync_copy(dst_hbm.at[pl.ds(wid * CPW, CPW)], dst_v.at[pl.ds(0, CPW)])
  plsc.subcore_barrier()

  def sfire(j, carry):
    pltpu.async_copy(ones_v, acc.at[dst_v.at[j]], ssem, add=True)
    return carry

  def sdrain(j, carry):
    pltpu.make_async_copy(ones_v, acc.at[dst_v.at[j]], ssem).wait()
    return carry

  lax.fori_loop(0, CPW, sfire, 0)
  lax.fori_loop(0, CPW, sdrain, 0)

  @pl.when(wid < EXTRA)
  def _():
    pltpu.sync_copy(dst_hbm.at[pl.ds(NW * CPW + wid, 1)],
                    dst_v.at[pl.ds(CPW, 1)])
    pltpu.sync_copy(ones_v, acc.at[dst_v.at[CPW]], add=True)

  plsc.subcore_barrier()
  pltpu.sync_copy(acc.at[pl.ds(rbase, RPT)],
                  out_hbm.at[pl.ds(cid * NP + rbase, RPT)])


def _rows_spec(k):
  return pl.BlockSpec((BN, k), lambda i: (i, 0))


def _full_spec(shape):
  return pl.BlockSpec(shape, lambda i: (0,) * len(shape))


def _dot(a, b):
  return jnp.dot(a, b, preferred_element_type=_f32,
                 precision=jax.lax.Precision.HIGHEST)


def _tc_call(body, ins, widths_in, out_widths):
  in_specs = []
  for a, kind in zip(ins, widths_in):
    if kind == "r":
      in_specs.append(_rows_spec(a.shape[1]))
    elif kind == "a":
      in_specs.append(pl.BlockSpec((BN, a.shape[1]), lambda i: (i, 0)))
    elif kind == "b":
      in_specs.append(pl.BlockSpec((BN, a.shape[1]),
                                   lambda i: (i + NP // BN, 0)))
    else:
      in_specs.append(_full_spec(a.shape))
  out_specs = [_rows_spec(w) for w in out_widths]
  out_shape = [jax.ShapeDtypeStruct((N, w), _f32) for w in out_widths]
  res = pl.pallas_call(
      body,
      grid=((N + BN - 1) // BN,),
      in_specs=in_specs,
      out_specs=out_specs,
      out_shape=out_shape,
  )(*ins)
  return res


def _u(da_ref, db_ref):
  return lax.rsqrt(1.0 + da_ref[...][:, 0:1] + db_ref[...][:, 0:1])


def _tca1(x_ref, w1_ref, h1_o):
  h1_o[...] = _dot(x_ref[...], w1_ref[...])


def _tca2(da_ref, db_ref, h1_ref, t1_o):
  t1_o[...] = _u(da_ref, db_ref) * h1_ref[...]


def _tcb(da_ref, db_ref, pa_ref, pb_ref, t1_ref, w2_ref, b1_ref, x1_o, t2_o):
  u = _u(da_ref, db_ref)
  x1 = jnp.tanh(u * (pa_ref[...] + pb_ref[...] + t1_ref[...]) + b1_ref[...])
  x1_o[...] = x1
  t2_o[...] = u * _dot(x1, w2_ref[...])


def _tcc(da_ref, db_ref, pa_ref, pb_ref, t2_ref, b2_ref, x2_o, t3_o):
  u = _u(da_ref, db_ref)
  x2 = jnp.tanh(u * (pa_ref[...] + pb_ref[...] + t2_ref[...]) + b2_ref[...])
  x2_o[...] = x2
  t3_o[...] = u * x2


def _tcd(da_ref, db_ref, pa_ref, pb_ref, t3_ref, w3_ref, b3_ref, w4_ref,
         z_o, pred_o, t4_o):
  u = _u(da_ref, db_ref)
  m = u * (pa_ref[...] + pb_ref[...] + t3_ref[...])
  z = jnp.tanh(_dot(m, w3_ref[...]) + b3_ref[...])
  z_o[...] = z
  zmax = jnp.max(z, axis=1, keepdims=True)
  ez = jnp.exp(z - zmax)
  pred_o[...] = ez / jnp.sum(ez, axis=1, keepdims=True)
  t4_o[...] = u * _dot(z, w4_ref[...])


def _tce(da_ref, db_ref, pa_ref, pb_ref, t4_ref, b4_ref, x2_ref, t5_o):
  u = _u(da_ref, db_ref)
  z2 = jnp.tanh(u * (pa_ref[...] + pb_ref[...] + t4_ref[...]) + b4_ref[...])
  z2 = z2 + x2_ref[...]
  t5_o[...] = u * z2


def _tcf(da_ref, db_ref, pa_ref, pb_ref, t5_ref, w5_ref, b5_ref, x1_ref, t6_o):
  u = _u(da_ref, db_ref)
  m = u * (pa_ref[...] + pb_ref[...] + t5_ref[...])
  z1 = jnp.tanh(_dot(m, w5_ref[...]) + b5_ref[...]) + x1_ref[...]
  t6_o[...] = u * z1


def _tcg(da_ref, db_ref, pa_ref, pb_ref, t6_ref, w6_ref, b6_ref, feat_o):
  u = _u(da_ref, db_ref)
  m = u * (pa_ref[...] + pb_ref[...] + t6_ref[...])
  feat_o[...] = jax.nn.sigmoid(_dot(m, w6_ref[...]) + b6_ref[...])


def kernel(x, edge_index, W1, b1, W2, b2, W3, b3, W4, b4, W5, b5, W6, b6):
  src = edge_index[0].reshape(CHUNKS, CH)
  dst = edge_index[1].reshape(CHUNKS, CH)
  zeros16 = jnp.zeros((NP, 16), _f32)
  zeros32 = jnp.zeros((NP, 32), _f32)
  ones = jnp.ones((CH, 16), _f32)

  degp = _deg_kernel(dst, ones, zeros16)
  h1, = _tc_call(_tca1, [x, W1], "rf", [32])

  t1, = _tc_call(_tca2, [degp, degp, h1], "abr", [32])

  g1 = _scat32(t1, src, dst, zeros32)
  x1, t2 = _tc_call(_tcb, [degp, degp, g1, g1, t1, W2, b1.reshape(1, -1)],
                    "ababrff", [32, 16])

  g2 = _scat16(t2, src, dst, zeros16)
  x2, t3 = _tc_call(_tcc, [degp, degp, g2, g2, t2, b2.reshape(1, -1)],
                    "ababrf", [16, 16])

  g3 = _scat16(t3, src, dst, zeros16)
  z, pred, t4 = _tc_call(
      _tcd, [degp, degp, g3, g3, t3, W3, b3.reshape(1, -1), W4],
      "ababrfff", [40, 40, 16])

  g4 = _scat16(t4, src, dst, zeros16)
  t5, = _tc_call(_tce, [degp, degp, g4, g4, t4, b4.reshape(1, -1), x2],
                 "ababrfr", [16])

  g5 = _scat16(t5, src, dst, zeros16)
  t6, = _tc_call(_tcf, [degp, degp, g5, g5, t5, W5, b5.reshape(1, -1), x1],
                 "ababrffr", [32])

  g6 = _scat32(t6, src, dst, zeros32)
  feat, = _tc_call(_tcg, [degp, degp, g6, g6, t6, W6, b6.reshape(1, -1)],
                   "ababrff", [128])

  return (feat, z, pred)


# speedup vs baseline: 45.5984x; 1.0003x over previous
import functools

import jax
import jax.numpy as jnp
from jax import lax
from jax.experimental import pallas as pl
from jax.experimental.pallas import tpu as pltpu
from jax.experimental.pallas import tpu_sc as plsc

N = 10000
E = 320000
NC = 2
NS = 16
NW = NC * NS
CH = 128
CHUNKS = E // CH
CPW = CHUNKS // NW
EXTRA = CHUNKS - CPW * NW
NP = 10240
RPT = NP // NS
BN = 2560

_MESH = plsc.VectorSubcoreMesh(core_axis_name="c", subcore_axis_name="s")
_f32 = jnp.float32


def _make_edge_scatter(w, k):
  rounds = CPW // k
  assert k * rounds == CPW

  @functools.partial(
      pl.kernel,
      out_type=jax.ShapeDtypeStruct((NP, 2 * w), _f32),
      mesh=_MESH,
      scratch_types=[
          pltpu.VMEM((CPW + 1, CH), jnp.int32),
          pltpu.VMEM((CPW + 1, CH), jnp.int32),
          pltpu.VMEM((2 * k * CH, w), _f32),
          pltpu.VMEM((CH, w), _f32),
          pltpu.VMEM_SHARED((NP, w), _f32),
          pltpu.SemaphoreType.DMA,
          pltpu.SemaphoreType.DMA,
          pltpu.SemaphoreType.DMA,
      ],
      compiler_params=pltpu.CompilerParams(use_tc_tiling_on_sc=False),
  )
  def kfn(t_hbm, src_hbm, dst_hbm, zeros_hbm, out_hbm, src_v, dst_v, rows_v,
          xrows_v, acc, gsem, ssem, xsem):
    cid = lax.axis_index("c")
    sid = lax.axis_index("s")
    wid = sid * NC + cid
    rbase = sid * RPT
    pltpu.sync_copy(zeros_hbm.at[pl.ds(rbase, RPT)],
                    acc.at[pl.ds(rbase, RPT)])
    cbase = wid * CPW
    pltpu.sync_copy(src_hbm.at[pl.ds(cbase, CPW)], src_v.at[pl.ds(0, CPW)])
    pltpu.sync_copy(dst_hbm.at[pl.ds(cbase, CPW)], dst_v.at[pl.ds(0, CPW)])

    @pl.when(wid < EXTRA)
    def _():
      pltpu.sync_copy(src_hbm.at[pl.ds(NW * CPW + wid, 1)],
                      src_v.at[pl.ds(CPW, 1)])
      pltpu.sync_copy(dst_hbm.at[pl.ds(NW * CPW + wid, 1)],
                      dst_v.at[pl.ds(CPW, 1)])
      pltpu.async_copy(t_hbm.at[src_v.at[CPW]], xrows_v, xsem)

    plsc.subcore_barrier()

    def gfire(j, carry, base=0, boff=0):
      pltpu.async_copy(t_hbm.at[src_v.at[base + j]],
                       rows_v.at[pl.ds((boff + j) * CH, CH)], gsem)
      return carry

    def gdrain(j, carry, base=0, boff=0):
      pltpu.make_async_copy(t_hbm.at[src_v.at[base + j]],
                            rows_v.at[pl.ds((boff + j) * CH, CH)],
                            gsem).wait()
      return carry

    def sfire(j, carry, base=0, boff=0):
      pltpu.async_copy(rows_v.at[pl.ds((boff + j) * CH, CH)],
                       acc.at[dst_v.at[base + j]], ssem, add=True)
      return carry

    def sdrain(j, carry, base=0, boff=0):
      pltpu.make_async_copy(rows_v.at[pl.ds((boff + j) * CH, CH)],
                            acc.at[dst_v.at[base + j]], ssem).wait()
      return carry

    for r in range(rounds):
      base, boff = r * k, (r % 2) * k
      if r >= 2:
        lax.fori_loop(0, k, functools.partial(sdrain, base=(r - 2) * k,
                                              boff=boff), 0)
      lax.fori_loop(0, k, functools.partial(gfire, base=base, boff=boff), 0)
      lax.fori_loop(0, k, functools.partial(gdrain, base=base, boff=boff), 0)
      lax.fori_loop(0, k, functools.partial(sfire, base=base, boff=boff), 0)
    for r in (max(rounds - 2, 0), rounds - 1):
      base, boff = r * k, (r % 2) * k
      lax.fori_loop(0, k, functools.partial(sdrain, base=base, boff=boff), 0)

    @pl.when(wid < EXTRA)
    def _():
      pltpu.make_async_copy(t_hbm.at[src_v.at[CPW]], xrows_v, xsem).wait()
      pltpu.async_copy(xrows_v, acc.at[dst_v.at[CPW]], xsem, add=True)
      pltpu.make_async_copy(xrows_v, acc.at[dst_v.at[CPW]], xsem).wait()

    plsc.subcore_barrier()

    @pl.when(cid == 0)
    def _():
      pltpu.sync_copy(acc.at[pl.ds(rbase, RPT)],
                      out_hbm.at[pl.ds(rbase, RPT), pl.ds(0, w)])

    @pl.when(cid == 1)
    def _():
      pltpu.sync_copy(acc.at[pl.ds(rbase, RPT)],
                      out_hbm.at[pl.ds(rbase, RPT), pl.ds(w, w)])

  return kfn


_scat16 = _make_edge_scatter(16, 13)
_scat32 = _make_edge_scatter(32, 6)


@functools.partial(
    pl.kernel,
    out_type=jax.ShapeDtypeStruct((NP, 32), _f32),
    mesh=_MESH,
    scratch_types=[
        pltpu.VMEM((CPW + 1, CH), jnp.int32),
        pltpu.VMEM((CH, 16), _f32),
        pltpu.VMEM_SHARED((NP, 16), _f32),
        pltpu.SemaphoreType.DMA,
    ],
    compiler_params=pltpu.CompilerParams(use_tc_tiling_on_sc=False),
)
def _deg_kernel(dst_hbm, ones_hbm, zeros_hbm, out_hbm, dst_v, ones_v, acc,
                ssem):
  cid = lax.axis_index("c")
  sid = lax.axis_index("s")
  wid = sid * NC + cid
  rbase = sid * RPT
  pltpu.sync_copy(ones_hbm, ones_v)
  pltpu.sync_copy(zeros_hbm.at[pl.ds(rbase, RPT)], acc.at[pl.ds(rbase, RPT)])
  pltpu.sync_copy(dst_hbm.at[pl.ds(wid * CPW, CPW)], dst_v.at[pl.ds(0, CPW)])
  plsc.subcore_barrier()

  def sfire(j, carry):
    pltpu.async_copy(ones_v, acc.at[dst_v.at[j]], ssem, add=True)
    return carry

  def sdrain(j, carry):
    pltpu.make_async_copy(ones_v, acc.at[dst_v.at[j]], ssem).wait()
    return carry

  lax.fori_loop(0, CPW, sfire, 0)
  lax.fori_loop(0, CPW, sdrain, 0)

  @pl.when(wid < EXTRA)
  def _():
    pltpu.sync_copy(dst_hbm.at[pl.ds(NW * CPW + wid, 1)],
                    dst_v.at[pl.ds(CPW, 1)])
    pltpu.sync_copy(ones_v, acc.at[dst_v.at[CPW]], add=True)

  plsc.subcore_barrier()

  @pl.when(cid == 0)
  def _():
    pltpu.sync_copy(acc.at[pl.ds(rbase, RPT)],
                    out_hbm.at[pl.ds(rbase, RPT), pl.ds(0, 16)])

  @pl.when(cid == 1)
  def _():
    pltpu.sync_copy(acc.at[pl.ds(rbase, RPT)],
                    out_hbm.at[pl.ds(rbase, RPT), pl.ds(16, 16)])


def _rows_spec(k):
  return pl.BlockSpec((BN, k), lambda i: (i, 0))


def _full_spec(shape):
  return pl.BlockSpec(shape, lambda i: (0,) * len(shape))


def _dot(a, b):
  return jnp.dot(a, b, preferred_element_type=_f32,
                 precision=jax.lax.Precision.HIGHEST)


def _tc_call(body, ins, kinds, out_widths):
  in_specs = []
  for a, kind in zip(ins, kinds):
    if kind in ("r", "p"):
      in_specs.append(_rows_spec(a.shape[1]))
    else:
      in_specs.append(_full_spec(a.shape))
  out_specs = [_rows_spec(w) for w in out_widths]
  out_shape = [jax.ShapeDtypeStruct((N, w), _f32) for w in out_widths]
  return pl.pallas_call(
      body,
      grid=((N + BN - 1) // BN,),
      in_specs=in_specs,
      out_specs=out_specs,
      out_shape=out_shape,
  )(*ins)


def _u(dd_ref):
  dd = dd_ref[...]
  return lax.rsqrt(1.0 + dd[:, 0:1] + dd[:, 16:17])


def _gsum(g_ref):
  g = g_ref[...]
  w = g.shape[1] // 2
  return g[:, :w] + g[:, w:]


def _tca1(x_ref, w1_ref, h1_o):
  h1_o[...] = _dot(x_ref[...], w1_ref[...])


def _tca2(dd_ref, h1_ref, t1_o):
  t1_o[...] = _u(dd_ref) * h1_ref[...]


def _tcb(dd_ref, g_ref, t1_ref, w2_ref, b1_ref, x1_o, t2_o):
  u = _u(dd_ref)
  x1 = jnp.tanh(u * (_gsum(g_ref) + t1_ref[...]) + b1_ref[...])
  x1_o[...] = x1
  t2_o[...] = u * _dot(x1, w2_ref[...])


def _tcc(dd_ref, g_ref, t2_ref, b2_ref, x2_o, t3_o):
  u = _u(dd_ref)
  x2 = jnp.tanh(u * (_gsum(g_ref) + t2_ref[...]) + b2_ref[...])
  x2_o[...] = x2
  t3_o[...] = u * x2


def _tcd(dd_ref, g_ref, t3_ref, w3_ref, b3_ref, w4_ref, z_o, pred_o, t4_o):
  u = _u(dd_ref)
  m = u * (_gsum(g_ref) + t3_ref[...])
  z = jnp.tanh(_dot(m, w3_ref[...]) + b3_ref[...])
  z_o[...] = z
  zmax = jnp.max(z, axis=1, keepdims=True)
  ez = jnp.exp(z - zmax)
  pred_o[...] = ez / jnp.sum(ez, axis=1, keepdims=True)
  t4_o[...] = u * _dot(z, w4_ref[...])


def _tce(dd_ref, g_ref, t4_ref, b4_ref, x2_ref, t5_o):
  u = _u(dd_ref)
  z2 = jnp.tanh(u * (_gsum(g_ref) + t4_ref[...]) + b4_ref[...])
  z2 = z2 + x2_ref[...]
  t5_o[...] = u * z2


def _tcf(dd_ref, g_ref, t5_ref, w5_ref, b5_ref, x1_ref, t6_o):
  u = _u(dd_ref)
  m = u * (_gsum(g_ref) + t5_ref[...])
  z1 = jnp.tanh(_dot(m, w5_ref[...]) + b5_ref[...]) + x1_ref[...]
  t6_o[...] = u * z1


def _tcg(dd_ref, g_ref, t6_ref, w6_ref, b6_ref, feat_o):
  u = _u(dd_ref)
  m = u * (_gsum(g_ref) + t6_ref[...])
  feat_o[...] = jax.nn.sigmoid(_dot(m, w6_ref[...]) + b6_ref[...])


def kernel(x, edge_index, W1, b1, W2, b2, W3, b3, W4, b4, W5, b5, W6, b6):
  src = edge_index[0].reshape(CHUNKS, CH)
  dst = edge_index[1].reshape(CHUNKS, CH)
  zeros16 = jnp.zeros((NP, 16), _f32)
  zeros32 = jnp.zeros((NP, 32), _f32)
  ones = jnp.ones((CH, 16), _f32)

  degp = _deg_kernel(dst, ones, zeros16)
  h1, = _tc_call(_tca1, [x, W1], "rf", [32])

  t1, = _tc_call(_tca2, [degp, h1], "pr", [32])

  g1 = _scat32(t1, src, dst, zeros32)
  x1, t2 = _tc_call(_tcb, [degp, g1, t1, W2, b1.reshape(1, -1)],
                    "pprff", [32, 16])

  g2 = _scat16(t2, src, dst, zeros16)
  x2, t3 = _tc_call(_tcc, [degp, g2, t2, b2.reshape(1, -1)],
                    "pprf", [16, 16])

  g3 = _scat16(t3, src, dst, zeros16)
  z, pred, t4 = _tc_call(
      _tcd, [degp, g3, t3, W3, b3.reshape(1, -1), W4],
      "pprfff", [40, 40, 16])

  g4 = _scat16(t4, src, dst, zeros16)
  t5, = _tc_call(_tce, [degp, g4, t4, b4.reshape(1, -1), x2],
                 "pprfr", [16])

  g5 = _scat16(t5, src, dst, zeros16)
  t6, = _tc_call(_tcf, [degp, g5, t5, W5, b5.reshape(1, -1), x1],
                 "pprffr", [32])

  g6 = _scat32(t6, src, dst, zeros32)
  feat, = _tc_call(_tcg, [degp, g6, t6, W6, b6.reshape(1, -1)],
                   "pprff", [128])

  return (feat, z, pred)
